# baseline (device time: 240304 ns/iter reference)
import jax
import jax.numpy as jnp
from jax import lax
from jax.experimental import pallas as pl
from jax.experimental.pallas import tpu as pltpu

MB_ = 4
NB = 6


def kernel(A, B):
    M, K = A.shape
    N = B.shape[1]
    RB = M // MB_
    CB = N // NB
    NTILES = MB_ * NB

    def body(
        a_ref, b_ref, recv_in_ref, out_ref, recv_hbm,
        a16_ref, a_stage, b_stage, b16_ref, recv_stage,
        a_sems, b_sems, r_sems, send_sems, recv_sems,
    ):
        del recv_in_ref
        my_x = lax.axis_index("x")
        my_y = lax.axis_index("y")
        peer = (my_x, 1 - my_y)

        def a_copy(i, slot):
            return pltpu.make_async_copy(
                a_ref.at[pl.ds(i * RB, RB), :], a_stage.at[slot], a_sems.at[slot]
            )

        def b_copy(j, slot):
            return pltpu.make_async_copy(
                b_ref.at[:, pl.ds(j * CB, CB)], b_stage.at[slot], b_sems.at[slot]
            )

        a_copy(0, 0).start()
        b_copy(0, 0).start()

        barrier = pltpu.get_barrier_semaphore()
        pl.semaphore_signal(
            barrier, inc=1, device_id=peer, device_id_type=pl.DeviceIdType.MESH
        )
        pl.semaphore_wait(barrier, 1)

        rdmas = {}
        for j in range(NB):
            with jax.named_scope(f"col#j={j}"):
                cs = pl.ds(j * CB, CB)
                b_copy(j, j % 2).wait()
                if j + 1 < NB:
                    b_copy(j + 1, (j + 1) % 2).start()
                b16_ref[j % 2] = b_stage[j % 2].astype(jnp.bfloat16)
                for i in range(MB_):
                    rs = pl.ds(i * RB, RB)
                    if j == 0:
                        a_copy(i, i % 2).wait()
                        if i + 1 < MB_:
                            a_copy(i + 1, (i + 1) % 2).start()
                        a16_ref[rs, :] = a_stage[i % 2].astype(jnp.bfloat16)
                    out_ref[rs, cs] = jnp.dot(
                        a16_ref[rs, :], b16_ref[j % 2],
                        preferred_element_type=jnp.float32,
                    ).astype(jnp.bfloat16)
                    t = j * MB_ + i
                    rdma = pltpu.make_async_remote_copy(
                        src_ref=out_ref.at[rs, cs],
                        dst_ref=recv_hbm.at[rs, cs],
                        send_sem=send_sems.at[t],
                        recv_sem=recv_sems.at[t],
                        device_id=peer,
                        device_id_type=pl.DeviceIdType.MESH,
                    )
                    rdma.start()
                    rdmas[t] = rdma

        for j in range(NB):
            for i in range(MB_):
                with jax.named_scope(f"add#t={j * MB_ + i}"):
                    cs = pl.ds(j * CB, CB)
                    rs = pl.ds(i * RB, RB)
                    t = j * MB_ + i
                    rdmas[t].wait_recv()
                    cp = pltpu.make_async_copy(
                        recv_hbm.at[rs, cs], recv_stage.at[t % 2], r_sems.at[t % 2]
                    )
                    cp.start()
                    cp.wait()
                    rdmas[t].wait_send()
                    out_ref[rs, cs] = (
                        out_ref[rs, cs].astype(jnp.float32)
                        + recv_stage[t % 2].astype(jnp.float32)
                    ).astype(jnp.bfloat16)

    out, _ = pl.pallas_call(
        body,
        out_shape=(
            jax.ShapeDtypeStruct((M, N), jnp.bfloat16),
            jax.ShapeDtypeStruct((M, N), jnp.bfloat16),
        ),
        in_specs=[
            pl.BlockSpec(memory_space=pltpu.MemorySpace.HBM),
            pl.BlockSpec(memory_space=pltpu.MemorySpace.HBM),
            pl.BlockSpec(memory_space=pltpu.MemorySpace.HBM),
        ],
        input_output_aliases={2: 1},
        out_specs=(
            pl.BlockSpec(memory_space=pltpu.MemorySpace.VMEM),
            pl.BlockSpec(memory_space=pltpu.MemorySpace.HBM),
        ),
        scratch_shapes=[
            pltpu.VMEM((M, K), jnp.bfloat16),
            pltpu.VMEM((2, RB, K), jnp.float32),
            pltpu.VMEM((2, K, CB), jnp.float32),
            pltpu.VMEM((2, K, CB), jnp.bfloat16),
            pltpu.VMEM((2, RB, CB), jnp.bfloat16),
            pltpu.SemaphoreType.DMA((2,)),
            pltpu.SemaphoreType.DMA((2,)),
            pltpu.SemaphoreType.DMA((2,)),
            pltpu.SemaphoreType.DMA((NTILES,)),
            pltpu.SemaphoreType.DMA((NTILES,)),
        ],
        compiler_params=pltpu.CompilerParams(
            vmem_limit_bytes=60 * 1024 * 1024,
            collective_id=0,
        ),
    )(A, B, jnp.zeros((M, N), jnp.bfloat16))
    return out


# device time: 230310 ns/iter; 1.0434x vs baseline; 1.0434x over previous
import jax
import jax.numpy as jnp
from jax import lax
from jax.experimental import pallas as pl
from jax.experimental.pallas import tpu as pltpu

MB_ = 4
NB = 8
NAC = 8


def kernel(A, B):
    M, K = A.shape
    N = B.shape[1]
    RB = M // MB_
    CB = N // NB
    RA = M // NAC
    NTILES = MB_ * NB

    def body(
        a_ref, b_ref, out_ref,
        a16_ref, a_stage, b_stage, b16_ref, recv_ref,
        a_sems, b_sems, send_sems, recv_sems,
    ):
        my_x = lax.axis_index("x")
        my_y = lax.axis_index("y")
        peer = (my_x, 1 - my_y)

        def a_copy(c):
            return pltpu.make_async_copy(
                a_ref.at[pl.ds(c * RA, RA), :], a_stage.at[c % 2], a_sems.at[c % 2]
            )

        def b_copy(j):
            return pltpu.make_async_copy(
                b_ref.at[:, pl.ds(j * CB, CB)], b_stage.at[j % 2], b_sems.at[j % 2]
            )

        a_copy(0).start()
        a_copy(1).start()
        b_copy(0).start()

        barrier = pltpu.get_barrier_semaphore()
        pl.semaphore_signal(
            barrier, inc=1, device_id=peer, device_id_type=pl.DeviceIdType.MESH
        )
        pl.semaphore_wait(barrier, 1)

        rdmas = {}
        for j in range(NB):
            cs = pl.ds(j * CB, CB)
            b_copy(j).wait()
            if j + 1 < NB:
                b_copy(j + 1).start()
            b16_ref[j % 2] = b_stage[j % 2].astype(jnp.bfloat16)
            for i in range(MB_):
                rs = pl.ds(i * RB, RB)
                if j == 0:
                    for c in (2 * i, 2 * i + 1):
                        a_copy(c).wait()
                        if c + 2 < NAC:
                            a_copy(c + 2).start()
                        a16_ref[pl.ds(c * RA, RA), :] = (
                            a_stage[c % 2].astype(jnp.bfloat16)
                        )
                out_ref[rs, cs] = jnp.dot(
                    a16_ref[rs, :], b16_ref[j % 2],
                    preferred_element_type=jnp.float32,
                ).astype(jnp.bfloat16)
                t = j * MB_ + i
                rdma = pltpu.make_async_remote_copy(
                    src_ref=out_ref.at[rs, cs],
                    dst_ref=recv_ref.at[rs, cs],
                    send_sem=send_sems.at[t],
                    recv_sem=recv_sems.at[t],
                    device_id=peer,
                    device_id_type=pl.DeviceIdType.MESH,
                )
                rdma.start()
                rdmas[t] = rdma

        for j in range(NB):
            cs = pl.ds(j * CB, CB)
            for i in range(MB_):
                rs = pl.ds(i * RB, RB)
                t = j * MB_ + i
                rdmas[t].wait_recv()
                rdmas[t].wait_send()
                out_ref[rs, cs] = (
                    out_ref[rs, cs].astype(jnp.float32)
                    + recv_ref[rs, cs].astype(jnp.float32)
                ).astype(jnp.bfloat16)

    return pl.pallas_call(
        body,
        out_shape=jax.ShapeDtypeStruct((M, N), jnp.bfloat16),
        in_specs=[
            pl.BlockSpec(memory_space=pltpu.MemorySpace.HBM),
            pl.BlockSpec(memory_space=pltpu.MemorySpace.HBM),
        ],
        out_specs=pl.BlockSpec(memory_space=pltpu.MemorySpace.VMEM),
        scratch_shapes=[
            pltpu.VMEM((M, K), jnp.bfloat16),
            pltpu.VMEM((2, RA, K), jnp.float32),
            pltpu.VMEM((2, K, CB), jnp.float32),
            pltpu.VMEM((2, K, CB), jnp.bfloat16),
            pltpu.VMEM((M, N), jnp.bfloat16),
            pltpu.SemaphoreType.DMA((2,)),
            pltpu.SemaphoreType.DMA((2,)),
            pltpu.SemaphoreType.DMA((NTILES,)),
            pltpu.SemaphoreType.DMA((NTILES,)),
        ],
        compiler_params=pltpu.CompilerParams(
            vmem_limit_bytes=63 * 1024 * 1024,
            collective_id=0,
        ),
    )(A, B)


# device time: 230160 ns/iter; 1.0441x vs baseline; 1.0007x over previous
import jax
import jax.numpy as jnp
from jax import lax
from jax.experimental import pallas as pl
from jax.experimental.pallas import tpu as pltpu

MB_ = 4
NB = 8
NAC = 8


def kernel(A, B):
    M, K = A.shape
    N = B.shape[1]
    RB = M // MB_
    CB = N // NB
    RA = M // NAC
    NTILES = MB_ * NB

    def body(
        a_ref, b_ref, out_ref,
        a16_ref, a_stage, b_stage, b16_ref, recv_ref,
        a_sems, b_sems, send_sems, recv_sems,
    ):
        my_x = lax.axis_index("x")
        my_y = lax.axis_index("y")
        peer = (my_x, 1 - my_y)

        def a_copy(c):
            return pltpu.make_async_copy(
                a_ref.at[pl.ds(c * RA, RA), :], a_stage.at[c % 2], a_sems.at[c % 2]
            )

        def b_copy(j):
            return pltpu.make_async_copy(
                b_ref.at[:, pl.ds(j * CB, CB)], b_stage.at[j % 2], b_sems.at[j % 2]
            )

        a_copy(0).start()
        a_copy(1).start()
        b_copy(0).start()

        barrier = pltpu.get_barrier_semaphore()
        pl.semaphore_signal(
            barrier, inc=1, device_id=peer, device_id_type=pl.DeviceIdType.MESH
        )
        pl.semaphore_wait(barrier, 1)

        rdmas = {}
        for j in range(NB):
            cs = pl.ds(j * CB, CB)
            b_copy(j).wait()
            if j + 1 < NB:
                b_copy(j + 1).start()
            b16_ref[j % 2] = b_stage[j % 2].astype(jnp.bfloat16)
            for i in range(MB_):
                rs = pl.ds(i * RB, RB)
                if j == 0:
                    for c in (2 * i, 2 * i + 1):
                        a_copy(c).wait()
                        a16_ref[pl.ds(c * RA, RA), :] = (
                            a_stage[c % 2].astype(jnp.bfloat16)
                        )
                        if c + 2 < NAC:
                            a_copy(c + 2).start()
                out_ref[rs, cs] = jnp.dot(
                    a16_ref[rs, :], b16_ref[j % 2],
                    preferred_element_type=jnp.float32,
                ).astype(jnp.bfloat16)
                t = j * MB_ + i
                rdma = pltpu.make_async_remote_copy(
                    src_ref=out_ref.at[rs, cs],
                    dst_ref=recv_ref.at[rs, cs],
                    send_sem=send_sems.at[t],
                    recv_sem=recv_sems.at[t],
                    device_id=peer,
                    device_id_type=pl.DeviceIdType.MESH,
                )
                rdma.start()
                rdmas[t] = rdma

        for j in range(NB):
            cs = pl.ds(j * CB, CB)
            for i in range(MB_):
                rs = pl.ds(i * RB, RB)
                t = j * MB_ + i
                rdmas[t].wait_recv()
                rdmas[t].wait_send()
                out_ref[rs, cs] = (
                    out_ref[rs, cs].astype(jnp.float32)
                    + recv_ref[rs, cs].astype(jnp.float32)
                ).astype(jnp.bfloat16)

    return pl.pallas_call(
        body,
        out_shape=jax.ShapeDtypeStruct((M, N), jnp.bfloat16),
        in_specs=[
            pl.BlockSpec(memory_space=pltpu.MemorySpace.HBM),
            pl.BlockSpec(memory_space=pltpu.MemorySpace.HBM),
        ],
        out_specs=pl.BlockSpec(memory_space=pltpu.MemorySpace.VMEM),
        scratch_shapes=[
            pltpu.VMEM((M, K), jnp.bfloat16),
            pltpu.VMEM((2, RA, K), jnp.float32),
            pltpu.VMEM((2, K, CB), jnp.float32),
            pltpu.VMEM((2, K, CB), jnp.bfloat16),
            pltpu.VMEM((M, N), jnp.bfloat16),
            pltpu.SemaphoreType.DMA((2,)),
            pltpu.SemaphoreType.DMA((2,)),
            pltpu.SemaphoreType.DMA((NTILES,)),
            pltpu.SemaphoreType.DMA((NTILES,)),
        ],
        compiler_params=pltpu.CompilerParams(
            vmem_limit_bytes=63 * 1024 * 1024,
            collective_id=0,
        ),
    )(A, B)


# device time: 224578 ns/iter; 1.0700x vs baseline; 1.0249x over previous
import jax
import jax.numpy as jnp
from jax import lax
from jax.experimental import pallas as pl
from jax.experimental.pallas import tpu as pltpu

MB_ = 4
NB = 8
NAC = 8


def kernel(A, B):
    M, K = A.shape
    N = B.shape[1]
    RB = M // MB_
    CB = N // NB
    RA = M // NAC
    NTILES = MB_ * NB

    def body(
        a_ref, b_ref, out_ref,
        a16_ref, a_stage, b_stage, b16_ref, part_ref, recv_ref, obuf_ref,
        a_sems, b_sems, o_sems, send_sems, recv_sems,
    ):
        my_x = lax.axis_index("x")
        my_y = lax.axis_index("y")
        peer = (my_x, 1 - my_y)

        def a_copy(c):
            return pltpu.make_async_copy(
                a_ref.at[pl.ds(c * RA, RA), :], a_stage.at[c % 2], a_sems.at[c % 2]
            )

        def b_copy(j):
            return pltpu.make_async_copy(
                b_ref.at[:, pl.ds(j * CB, CB)], b_stage.at[j % 2], b_sems.at[j % 2]
            )

        def o_copy(t):
            j, i = divmod(t, MB_)
            return pltpu.make_async_copy(
                obuf_ref.at[t % 2],
                out_ref.at[pl.ds(i * RB, RB), pl.ds(j * CB, CB)],
                o_sems.at[t % 2],
            )

        a_copy(0).start()
        a_copy(1).start()
        b_copy(0).start()

        barrier = pltpu.get_barrier_semaphore()
        pl.semaphore_signal(
            barrier, inc=1, device_id=peer, device_id_type=pl.DeviceIdType.MESH
        )
        pl.semaphore_wait(barrier, 1)

        rdmas = {}
        for j in range(NB):
            b_copy(j).wait()
            if j + 1 < NB:
                b_copy(j + 1).start()
            b16_ref[j % 2] = b_stage[j % 2].astype(jnp.bfloat16)
            for i in range(MB_):
                rs = pl.ds(i * RB, RB)
                if j == 0:
                    for c in (2 * i, 2 * i + 1):
                        a_copy(c).wait()
                        a16_ref[pl.ds(c * RA, RA), :] = (
                            a_stage[c % 2].astype(jnp.bfloat16)
                        )
                        if c + 2 < NAC:
                            a_copy(c + 2).start()
                part_ref[j, rs, :] = jnp.dot(
                    a16_ref[rs, :], b16_ref[j % 2],
                    preferred_element_type=jnp.float32,
                ).astype(jnp.bfloat16)
                t = j * MB_ + i
                rdma = pltpu.make_async_remote_copy(
                    src_ref=part_ref.at[j, rs, :],
                    dst_ref=recv_ref.at[j, rs, :],
                    send_sem=send_sems.at[t],
                    recv_sem=recv_sems.at[t],
                    device_id=peer,
                    device_id_type=pl.DeviceIdType.MESH,
                )
                rdma.start()
                rdmas[t] = rdma

        for j in range(NB):
            for i in range(MB_):
                rs = pl.ds(i * RB, RB)
                t = j * MB_ + i
                rdmas[t].wait_recv()
                rdmas[t].wait_send()
                if t >= 2:
                    o_copy(t - 2).wait()
                obuf_ref[t % 2] = (
                    part_ref[j, rs, :].astype(jnp.float32)
                    + recv_ref[j, rs, :].astype(jnp.float32)
                ).astype(jnp.bfloat16)
                o_copy(t).start()
        o_copy(NTILES - 2).wait()
        o_copy(NTILES - 1).wait()

    return pl.pallas_call(
        body,
        out_shape=jax.ShapeDtypeStruct((M, N), jnp.bfloat16),
        in_specs=[
            pl.BlockSpec(memory_space=pltpu.MemorySpace.HBM),
            pl.BlockSpec(memory_space=pltpu.MemorySpace.HBM),
        ],
        out_specs=pl.BlockSpec(memory_space=pltpu.MemorySpace.HBM),
        scratch_shapes=[
            pltpu.VMEM((M, K), jnp.bfloat16),
            pltpu.VMEM((2, RA, K), jnp.float32),
            pltpu.VMEM((2, K, CB), jnp.float32),
            pltpu.VMEM((2, K, CB), jnp.bfloat16),
            pltpu.VMEM((NB, M, CB), jnp.bfloat16),
            pltpu.VMEM((NB, M, CB), jnp.bfloat16),
            pltpu.VMEM((2, RB, CB), jnp.bfloat16),
            pltpu.SemaphoreType.DMA((2,)),
            pltpu.SemaphoreType.DMA((2,)),
            pltpu.SemaphoreType.DMA((2,)),
            pltpu.SemaphoreType.DMA((NTILES,)),
            pltpu.SemaphoreType.DMA((NTILES,)),
        ],
        compiler_params=pltpu.CompilerParams(
            vmem_limit_bytes=63 * 1024 * 1024,
            collective_id=0,
        ),
    )(A, B)
